# baseline (speedup 1.0000x reference)
import functools

import jax
import jax.numpy as jnp
from jax import lax
from jax.experimental import pallas as pl
from jax.experimental.pallas import tpu as pltpu
from jax.experimental.pallas import tpu_sc as plsc

N_ACT = 1000001
N_RES = 100001
NB = 4096
NS = 200
B = NB * NS
F = 18
D = 32
OUT_D = 80
NW = 32
B_PER_W = NB // NW
BC = 16
NCHUNK = B_PER_W // BC
CHUNK = BC * NS
SUB = 400
NSUB = CHUNK // SUB
L = 16


NS_H = NS // 2
CHUNK_H = BC * NS_H
NSUB_H = CHUNK_H // SUB


def _make_sc_kernel(s0):
    mesh = plsc.VectorSubcoreMesh(core_axis_name="c", subcore_axis_name="s")

    @functools.partial(
        pl.kernel,
        mesh=mesh,
        compiler_params=pltpu.CompilerParams(
            use_tc_tiling_on_sc=False, needs_layout_passes=False),
        out_type=jax.ShapeDtypeStruct((B // 2, OUT_D), jnp.float32),
        scratch_types=[
            pltpu.VMEM((2, NS_H, BC), jnp.float32),
            pltpu.VMEM((F - 2, NS_H, BC), jnp.float32),
            pltpu.VMEM((CHUNK_H,), jnp.int32),
            pltpu.VMEM((CHUNK_H,), jnp.int32),
            pltpu.VMEM((2, SUB, D), jnp.float32),
            pltpu.VMEM((2, SUB, D), jnp.float32),
            pltpu.VMEM((SUB, F - 2), jnp.float32),
            pltpu.SemaphoreType.DMA,
            pltpu.SemaphoreType.DMA,
            pltpu.SemaphoreType.DMA,
            pltpu.SemaphoreType.DMA,
            pltpu.SemaphoreType.DMA,
            pltpu.SemaphoreType.DMA,
        ],
    )
    def sc_kernel(dataT, act2, res2, out_hbm,
                  id_v, f_v, ai_v, ri_v, a_v, r_v, f2_v,
                  sem_a0, sem_a1, sem_r0, sem_r1, sem_w0, sem_w1):
        wid = lax.axis_index("s") * 2 + lax.axis_index("c")
        lanes = lax.iota(jnp.int32, L)
        sems_a = (sem_a0, sem_a1)
        sems_r = (sem_r0, sem_r1)
        sems_w = (sem_w0, sem_w1)

        def gathers(k, buf):
            sk = pl.ds(k * SUB, SUB)
            cp_a = pltpu.make_async_copy(
                act2.at[ai_v.at[sk]], a_v.at[buf], sems_a[buf])
            cp_a.start()
            cp_r = pltpu.make_async_copy(
                res2.at[ri_v.at[sk]], r_v.at[buf], sems_r[buf])
            cp_r.start()
            return cp_a, cp_r

        def chunk_body(cj, carry):
            b0 = wid * B_PER_W + cj * BC
            r0 = b0 * NS_H
            pltpu.sync_copy(
                dataT.at[pl.ds(0, 2), pl.ds(s0, NS_H), pl.ds(b0, BC)], id_v)
            pltpu.sync_copy(
                dataT.at[pl.ds(2, F - 2), pl.ds(s0, NS_H), pl.ds(b0, BC)],
                f_v)

            def idx_body(s, icarry):
                pos = lanes * NS_H + s
                va = id_v[0, s, pl.ds(0, L)].astype(jnp.int32)
                vr = id_v[1, s, pl.ds(0, L)].astype(jnp.int32)
                plsc.store_scatter(ai_v, [pos], va)
                plsc.store_scatter(ri_v, [pos], vr)
                return icarry

            lax.fori_loop(0, NS_H, idx_body, 0, unroll=2)

            cps = gathers(0, 0)
            writes = []
            for k in range(NSUB_H):
                buf = k % 2
                if k + 1 < NSUB_H:
                    nxt = gathers(k + 1, (k + 1) % 2)

                for j2 in range(SUB // NS_H):
                    j = k * (SUB // NS_H) + j2
                    jvec = jnp.full((L,), j, jnp.int32)

                    def feat_body(s, fcarry, j2=j2, jvec=jvec):
                        g = plsc.load_gather(
                            f_v, [lanes, jnp.full((L,), s, jnp.int32), jvec])
                        f2_v[j2 * NS_H + s, pl.ds(0, L)] = g
                        return fcarry

                    lax.fori_loop(0, NS_H, feat_body, 0, unroll=8)
                cps[0].wait()
                cps[1].wait()
                rows = pl.ds(r0 + k * SUB, SUB)
                wa = pltpu.make_async_copy(
                    a_v.at[buf], out_hbm.at[rows, pl.ds(0, D)], sems_w[buf])
                wa.start()
                wr = pltpu.make_async_copy(
                    r_v.at[buf], out_hbm.at[rows, pl.ds(D, D)], sems_w[buf])
                wr.start()
                pltpu.sync_copy(
                    f2_v, out_hbm.at[rows, pl.ds(2 * D, F - 2)])
                if k + 1 < NSUB_H:
                    cps = nxt
                writes.append((wa, wr))
                if len(writes) >= 2:
                    wpa, wpr = writes.pop(0)
                    wpa.wait()
                    wpr.wait()
            for wpa, wpr in writes:
                wpa.wait()
                wpr.wait()
            return carry

        lax.fori_loop(0, NCHUNK, chunk_body, 0)

    return sc_kernel


_sc_kernel_a = _make_sc_kernel(0)
_sc_kernel_b = _make_sc_kernel(NS_H)


def kernel(data, act_table, res_table):
    dataT = data.transpose(2, 1, 0)
    oa = _sc_kernel_a(dataT, act_table, res_table)
    ob = _sc_kernel_b(dataT, act_table, res_table)
    return jnp.concatenate(
        [oa.reshape(NB, NS_H, OUT_D), ob.reshape(NB, NS_H, OUT_D)], axis=1)


# speedup vs baseline: 1.6004x; 1.6004x over previous
import functools

import jax
import jax.numpy as jnp
from jax import lax
from jax.experimental import pallas as pl
from jax.experimental.pallas import tpu as pltpu
from jax.experimental.pallas import tpu_sc as plsc

N_ACT = 1000001
N_RES = 100001
NB = 4096
NS = 200
B = NB * NS
F = 18
D = 32
OUT_D = 80
NW = 32
B_PER_W = NB // NW
BC = 16
NCHUNK = B_PER_W // BC
CHUNK = BC * NS
SUB = 400
NSUB = CHUNK // SUB
L = 16


def _make_sc_kernel():
    mesh = plsc.VectorSubcoreMesh(core_axis_name="c", subcore_axis_name="s")

    @functools.partial(
        pl.kernel,
        mesh=mesh,
        compiler_params=pltpu.CompilerParams(
            use_tc_tiling_on_sc=False, needs_layout_passes=False),
        out_type=jax.ShapeDtypeStruct((B, OUT_D), jnp.float32),
        scratch_types=[
            pltpu.VMEM((2, NS, BC), jnp.float32),
            pltpu.VMEM((F - 2, NS, BC), jnp.float32),
            pltpu.VMEM((CHUNK,), jnp.int32),
            pltpu.VMEM((CHUNK,), jnp.int32),
            pltpu.VMEM((2, SUB, D), jnp.float32),
            pltpu.VMEM((2, SUB, D), jnp.float32),
            pltpu.VMEM((SUB, F - 2), jnp.float32),
            pltpu.SemaphoreType.DMA,
            pltpu.SemaphoreType.DMA,
            pltpu.SemaphoreType.DMA,
            pltpu.SemaphoreType.DMA,
            pltpu.SemaphoreType.DMA,
            pltpu.SemaphoreType.DMA,
        ],
    )
    def sc_kernel(dataT, act2, res2, out_hbm,
                  id_v, f_v, ai_v, ri_v, a_v, r_v, f2_v,
                  sem_a0, sem_a1, sem_r0, sem_r1, sem_w0, sem_w1):
        wid = lax.axis_index("s") * 2 + lax.axis_index("c")
        lanes = lax.iota(jnp.int32, L)
        sems_a = (sem_a0, sem_a1)
        sems_r = (sem_r0, sem_r1)
        sems_w = (sem_w0, sem_w1)

        def gathers(k, buf):
            sk = pl.ds(k * SUB, SUB)
            cp_a = pltpu.make_async_copy(
                act2.at[ai_v.at[sk]], a_v.at[buf], sems_a[buf])
            cp_a.start()
            cp_r = pltpu.make_async_copy(
                res2.at[ri_v.at[sk]], r_v.at[buf], sems_r[buf])
            cp_r.start()
            return cp_a, cp_r

        def chunk_body(cj, carry):
            b0 = wid * B_PER_W + cj * BC
            r0 = b0 * NS
            pltpu.sync_copy(dataT.at[pl.ds(0, 2), :, pl.ds(b0, BC)], id_v)
            pltpu.sync_copy(dataT.at[pl.ds(2, F - 2), :, pl.ds(b0, BC)], f_v)

            def idx_body(s, icarry):
                pos = lanes * NS + s
                va = id_v[0, s, pl.ds(0, L)].astype(jnp.int32)
                vr = id_v[1, s, pl.ds(0, L)].astype(jnp.int32)
                plsc.store_scatter(ai_v, [pos], va)
                plsc.store_scatter(ri_v, [pos], vr)
                return icarry

            lax.fori_loop(0, NS, idx_body, 0, unroll=2)

            cps = gathers(0, 0)
            writes = []
            for k in range(NSUB):
                buf = k % 2
                if k + 1 < NSUB:
                    nxt = gathers(k + 1, (k + 1) % 2)

                for j2 in range(SUB // NS):
                    j = k * (SUB // NS) + j2
                    jvec = jnp.full((L,), j, jnp.int32)

                    def feat_body(s, fcarry, j2=j2, jvec=jvec):
                        g = plsc.load_gather(
                            f_v, [lanes, jnp.full((L,), s, jnp.int32), jvec])
                        f2_v[j2 * NS + s, pl.ds(0, L)] = g
                        return fcarry

                    lax.fori_loop(0, NS, feat_body, 0, unroll=8)
                cps[0].wait()
                cps[1].wait()
                rows = pl.ds(r0 + k * SUB, SUB)
                wa = pltpu.make_async_copy(
                    a_v.at[buf], out_hbm.at[rows, pl.ds(0, D)], sems_w[buf])
                wa.start()
                wr = pltpu.make_async_copy(
                    r_v.at[buf], out_hbm.at[rows, pl.ds(D, D)], sems_w[buf])
                wr.start()
                pltpu.sync_copy(
                    f2_v, out_hbm.at[rows, pl.ds(2 * D, F - 2)])
                if k + 1 < NSUB:
                    cps = nxt
                writes.append((wa, wr))
                if len(writes) >= 2:
                    wpa, wpr = writes.pop(0)
                    wpa.wait()
                    wpr.wait()
            for wpa, wpr in writes:
                wpa.wait()
                wpr.wait()
            return carry

        lax.fori_loop(0, NCHUNK, chunk_body, 0)

    return sc_kernel


_sc_kernel = _make_sc_kernel()


def kernel(data, act_table, res_table):
    dataT = data.transpose(2, 1, 0)
    out = _sc_kernel(dataT, act_table, res_table)
    return out.reshape(NB, NS, OUT_D)
